# baseline (speedup 1.0000x reference)
import jax
import jax.numpy as jnp
from jax.experimental import pallas as pl
from jax.experimental.pallas import tpu as pltpu

_BB = 4


def _bmm_kernel(a_ref, h_ref, o_ref):
    for j in range(_BB):
        o_ref[j] = jnp.dot(a_ref[j], h_ref[j], preferred_element_type=jnp.float32)


@jax.jit
def kernel(h_text, alignment):
    B, Lm, Lt = alignment.shape
    Ht = h_text.shape[2]
    grid = (B // _BB,)
    return pl.pallas_call(
        _bmm_kernel,
        grid=grid,
        in_specs=[
            pl.BlockSpec((_BB, Lm, Lt), lambda b: (b, 0, 0)),
            pl.BlockSpec((_BB, Lt, Ht), lambda b: (b, 0, 0)),
        ],
        out_specs=pl.BlockSpec((_BB, Lm, Ht), lambda b: (b, 0, 0)),
        out_shape=jax.ShapeDtypeStruct((B, Lm, Ht), jnp.float32),
        compiler_params=pltpu.CompilerParams(
            dimension_semantics=("arbitrary",),
        ),
    )(alignment, h_text)


# speedup vs baseline: 1.0577x; 1.0577x over previous
import jax
import jax.numpy as jnp
from jax.experimental import pallas as pl
from jax.experimental.pallas import tpu as pltpu

_BB = 2


def _bmm_kernel(a_ref, h_ref, o_ref):
    for j in range(_BB):
        o_ref[j] = jnp.dot(
            a_ref[j].astype(jnp.bfloat16),
            h_ref[j].astype(jnp.bfloat16),
            preferred_element_type=jnp.float32,
        )


@jax.jit
def kernel(h_text, alignment):
    B, Lm, Lt = alignment.shape
    Ht = h_text.shape[2]
    grid = (B // _BB,)
    return pl.pallas_call(
        _bmm_kernel,
        grid=grid,
        in_specs=[
            pl.BlockSpec((_BB, Lm, Lt), lambda b: (b, 0, 0)),
            pl.BlockSpec((_BB, Lt, Ht), lambda b: (b, 0, 0)),
        ],
        out_specs=pl.BlockSpec((_BB, Lm, Ht), lambda b: (b, 0, 0)),
        out_shape=jax.ShapeDtypeStruct((B, Lm, Ht), jnp.float32),
        compiler_params=pltpu.CompilerParams(
            dimension_semantics=("arbitrary",),
        ),
    )(alignment, h_text)
